# Initial kernel scaffold; baseline (speedup 1.0000x reference)
import jax
import jax.numpy as jnp
from jax.experimental import pallas as pl


def kernel(embed_feat, h_gru, ei_follows, ei_likes, Win_f0, Win_f1, Wa_f0, Wa_f1, Win_l0, Win_l1, Wa_l0, Wa_l1, Wp1, bp1, Wp2):
    raise NotImplementedError("write your pallas kernel here")


# speedup vs baseline: 17.6897x; 17.6897x over previous
import dataclasses
import functools

import jax
import jax.numpy as jnp
from jax import lax
from jax.experimental import pallas as pl
from jax.experimental.pallas import tpu as pltpu
from jax.experimental.pallas import tpu_sc as plsc

N = 10000
E = 160000
OUT = 64
ROW = 144
ADW = 16
NSUB = 16
EPS = E // NSUB
CH = 80
NCHUNK = EPS // CH
NPAD = 10240
RPS = NPAD // NSUB
BLK = 200


def _prep_body(e_ref, h_ref, w_ref, v_ref, zs_ref, ad_ref):
    x = jnp.concatenate([e_ref[...], h_ref[...]], axis=1)
    z = lax.dot_general(x, w_ref[...], (((1,), (1,)), ((), ())),
                        preferred_element_type=jnp.float32)
    a = lax.dot_general(x, v_ref[...], (((1,), (0,)), ((), ())),
                        preferred_element_type=jnp.float32)
    pad = jnp.zeros((BLK, ROW - 2 * OUT - 2), jnp.float32)
    zs_f = jnp.concatenate([z[:, 0:64], z[:, 64:128], a[:, 0:1], a[:, 1:2], pad], axis=1)
    zs_l = jnp.concatenate([z[:, 128:192], z[:, 192:256], a[:, 2:3], a[:, 3:4], pad], axis=1)
    zs_ref[...] = jnp.stack([zs_f, zs_l], axis=0)
    ad_ref[...] = jnp.concatenate(
        [a[:, 4:8], jnp.zeros((BLK, ADW - 4), jnp.float32)], axis=1)


_prep = pl.pallas_call(
    _prep_body,
    grid=(N // BLK,),
    in_specs=[
        pl.BlockSpec((BLK, 64), lambda i: (i, 0)),
        pl.BlockSpec((BLK, 64), lambda i: (i, 0)),
        pl.BlockSpec((256, 128), lambda i: (0, 0)),
        pl.BlockSpec((128, 8), lambda i: (0, 0)),
    ],
    out_specs=[
        pl.BlockSpec((2, BLK, ROW), lambda i: (0, i, 0)),
        pl.BlockSpec((BLK, ADW), lambda i: (i, 0)),
    ],
    out_shape=[
        jax.ShapeDtypeStruct((2, N, ROW), jnp.float32),
        jax.ShapeDtypeStruct((N, ADW), jnp.float32),
    ],
)


def _sc_body(zs_hbm, ad_hbm, src_hbm, dst_hbm, zero_hbm, out_hbm,
             idx_s, idx_d, rows_v, ad_v, ex0_v, ex1_v, acc):
    c = lax.axis_index("c")
    s = lax.axis_index("s")

    pltpu.sync_copy(zero_hbm, acc.at[pl.ds(s * RPS, RPS)])
    plsc.subcore_barrier()

    iota16 = lax.broadcasted_iota(jnp.int32, (16,), 0)
    cN = c * N
    col_s0 = jnp.broadcast_to(jnp.int32(2 * OUT), (16,))
    col_s1 = col_s0 + 1
    col_d0 = jnp.broadcast_to(2 * c, (16,))
    col_d1 = col_d0 + 1

    @pl.loop(0, NCHUNK)
    def _chunk(k):
        base = c * E + s * EPS + k * CH
        pltpu.sync_copy(src_hbm.at[pl.ds(base, CH)], idx_s)
        pltpu.sync_copy(dst_hbm.at[pl.ds(base, CH)], idx_d)
        for j in range(CH // 16):
            idx_s[pl.ds(j * 16, 16)] = idx_s[pl.ds(j * 16, 16)] + cN
        pltpu.sync_copy(zs_hbm.at[idx_s], rows_v)
        pltpu.sync_copy(ad_hbm.at[idx_d], ad_v)
        for j in range(CH // 16):
            ridx = iota16 + j * 16
            a_s0 = plsc.load_gather(rows_v, [ridx, col_s0])
            a_s1 = plsc.load_gather(rows_v, [ridx, col_s1])
            a_d0 = plsc.load_gather(ad_v, [ridx, col_d0])
            a_d1 = plsc.load_gather(ad_v, [ridx, col_d1])
            e0 = a_s0 + a_d0
            e1 = a_s1 + a_d1
            ex0 = jnp.exp(jnp.maximum(e0, 0.01 * e0))
            ex1 = jnp.exp(jnp.maximum(e1, 0.01 * e1))
            plsc.store_scatter(rows_v, [ridx, col_s0], ex0)
            plsc.store_scatter(rows_v, [ridx, col_s1], ex1)
            ex0_v[pl.ds(j * 16, 16)] = ex0
            ex1_v[pl.ds(j * 16, 16)] = ex1

        @pl.loop(0, CH)
        def _scale(i):
            bi = jnp.broadcast_to(i, (16,))
            w0 = plsc.load_gather(ex0_v, [bi])
            w1 = plsc.load_gather(ex1_v, [bi])
            for cb in range(4):
                rows_v[i, pl.ds(cb * 16, 16)] = rows_v[i, pl.ds(cb * 16, 16)] * w0
            for cb in range(4, 8):
                rows_v[i, pl.ds(cb * 16, 16)] = rows_v[i, pl.ds(cb * 16, 16)] * w1

        pltpu.sync_copy(rows_v, acc.at[idx_d], add=True)

    plsc.subcore_barrier()
    pltpu.sync_copy(acc.at[pl.ds(s * RPS, RPS)],
                    out_hbm.at[c, pl.ds(s * RPS, RPS)])


_sc_params = pltpu.CompilerParams(use_tc_tiling_on_sc=False)
if "needs_layout_passes" in pltpu.CompilerParams.__dataclass_fields__:
    _sc_params = dataclasses.replace(_sc_params, needs_layout_passes=False)

_sc_edges = functools.partial(
    pl.kernel,
    out_type=jax.ShapeDtypeStruct((2, NPAD, ROW), jnp.float32),
    mesh=plsc.VectorSubcoreMesh(core_axis_name="c", subcore_axis_name="s"),
    compiler_params=_sc_params,
    scratch_types=[
        pltpu.VMEM((CH,), jnp.int32),
        pltpu.VMEM((CH,), jnp.int32),
        pltpu.VMEM((CH, ROW), jnp.float32),
        pltpu.VMEM((CH, ADW), jnp.float32),
        pltpu.VMEM((CH,), jnp.float32),
        pltpu.VMEM((CH,), jnp.float32),
        pltpu.VMEM_SHARED((NPAD, ROW), jnp.float32),
    ],
)(_sc_body)


def _heads(acc, g):
    num = acc[g, :, 0:128]
    d0 = acc[g, :, 128]
    d1 = acc[g, :, 129]
    d0 = jnp.where(d0 > 0, d0, 1.0)
    d1 = jnp.where(d1 > 0, d1, 1.0)
    den = jnp.concatenate(
        [jnp.broadcast_to(d0[:, None], (BLK, OUT)),
         jnp.broadcast_to(d1[:, None], (BLK, OUT))], axis=1)
    return num / den


def _wsum_body(acc_ref, wp1_ref, bp1_ref, wp2_ref, wsum_ref):
    acc = acc_ref[...]
    h_f = _heads(acc, 0)
    h_l = _heads(acc, 1)

    def logit_sum(h):
        t = jnp.tanh(
            lax.dot_general(h, wp1_ref[...], (((1,), (1,)), ((), ())),
                            preferred_element_type=jnp.float32) + bp1_ref[...])
        w = lax.dot_general(t, wp2_ref[...], (((1,), (1,)), ((), ())),
                            preferred_element_type=jnp.float32)
        return jnp.sum(w)

    pf = logit_sum(h_f)
    plv = logit_sum(h_l)
    row = lax.broadcasted_iota(jnp.int32, (8, 128), 0)
    col = lax.broadcasted_iota(jnp.int32, (8, 128), 1)
    upd = (jnp.where((row == 0) & (col == 0), pf, 0.0)
           + jnp.where((row == 0) & (col == 1), plv, 0.0))

    @pl.when(pl.program_id(0) == 0)
    def _():
        wsum_ref[...] = jnp.zeros((8, 128), jnp.float32)

    wsum_ref[...] += upd


_wsum = pl.pallas_call(
    _wsum_body,
    grid=(N // BLK,),
    in_specs=[
        pl.BlockSpec((2, BLK, ROW), lambda i: (0, i, 0)),
        pl.BlockSpec((128, 128), lambda i: (0, 0)),
        pl.BlockSpec((1, 128), lambda i: (0, 0)),
        pl.BlockSpec((1, 128), lambda i: (0, 0)),
    ],
    out_specs=pl.BlockSpec((8, 128), lambda i: (0, 0)),
    out_shape=jax.ShapeDtypeStruct((8, 128), jnp.float32),
)


def _combine_body(acc_ref, wsum_ref, out_ref):
    s0 = wsum_ref[0, 0] / N
    s1 = wsum_ref[0, 1] / N
    m = jnp.maximum(s0, s1)
    b0 = jnp.exp(s0 - m)
    b1 = jnp.exp(s1 - m)
    beta0 = b0 / (b0 + b1)
    beta1 = b1 / (b0 + b1)
    acc = acc_ref[...]
    out_ref[...] = beta0 * _heads(acc, 0) + beta1 * _heads(acc, 1)


_combine = pl.pallas_call(
    _combine_body,
    grid=(N // BLK,),
    in_specs=[
        pl.BlockSpec((2, BLK, ROW), lambda i: (0, i, 0)),
        pl.BlockSpec((8, 128), lambda i: (0, 0)),
    ],
    out_specs=pl.BlockSpec((BLK, 128), lambda i: (i, 0)),
    out_shape=jax.ShapeDtypeStruct((N, 128), jnp.float32),
)


def kernel(embed_feat, h_gru, ei_follows, ei_likes, Win_f0, Win_f1, Wa_f0,
           Wa_f1, Win_l0, Win_l1, Wa_l0, Wa_l1, Wp1, bp1, Wp2):
    w_all = jnp.concatenate([Win_f0, Win_f1, Win_l0, Win_l1], axis=0)
    folds = []
    for win, wa in ((Win_f0, Wa_f0), (Win_f1, Wa_f1), (Win_l0, Wa_l0), (Win_l1, Wa_l1)):
        folds.append(win.T @ wa[0, :OUT])
    for win, wa in ((Win_f0, Wa_f0), (Win_f1, Wa_f1), (Win_l0, Wa_l0), (Win_l1, Wa_l1)):
        folds.append(win.T @ wa[0, OUT:])
    v_fold = jnp.stack(folds, axis=1)
    src_all = jnp.concatenate([ei_follows[0], ei_likes[0]])
    dst_all = jnp.concatenate([ei_follows[1], ei_likes[1]])
    zeros_h = jnp.zeros((RPS, ROW), jnp.float32)

    zs3, ad = _prep(embed_feat, h_gru, w_all, v_fold)
    zs_all = zs3.reshape(2 * N, ROW)
    acc3 = _sc_edges(zs_all, ad, src_all, dst_all, zeros_h)
    wsum = _wsum(acc3, Wp1, bp1.reshape(1, 128), Wp2)
    return _combine(acc3, wsum)


# speedup vs baseline: 20.8337x; 1.1777x over previous
import dataclasses
import functools

import jax
import jax.numpy as jnp
from jax import lax
from jax.experimental import pallas as pl
from jax.experimental.pallas import tpu as pltpu
from jax.experimental.pallas import tpu_sc as plsc

N = 10000
E = 160000
OUT = 64
ROW = 144
ADW = 16
NSUB = 16
EPS = E // NSUB
CH = 80
NCHUNK = EPS // CH
NPAD = 10240
RPS = NPAD // NSUB
BLK = 200


def _prep_body(e_ref, h_ref, w_ref, v_ref, zs_ref, ad_ref):
    x = jnp.concatenate([e_ref[...], h_ref[...]], axis=1)
    z = lax.dot_general(x, w_ref[...], (((1,), (1,)), ((), ())),
                        preferred_element_type=jnp.float32)
    a = lax.dot_general(x, v_ref[...], (((1,), (0,)), ((), ())),
                        preferred_element_type=jnp.float32)
    pad = jnp.zeros((BLK, ROW - 2 * OUT - 2), jnp.float32)
    zs_f = jnp.concatenate([z[:, 0:64], z[:, 64:128], a[:, 0:1], a[:, 1:2], pad], axis=1)
    zs_l = jnp.concatenate([z[:, 128:192], z[:, 192:256], a[:, 2:3], a[:, 3:4], pad], axis=1)
    zs_ref[...] = jnp.stack([zs_f, zs_l], axis=0)
    ad_ref[...] = jnp.concatenate(
        [a[:, 4:8], jnp.zeros((BLK, ADW - 4), jnp.float32)], axis=1)


_prep = pl.pallas_call(
    _prep_body,
    grid=(N // BLK,),
    in_specs=[
        pl.BlockSpec((BLK, 64), lambda i: (i, 0)),
        pl.BlockSpec((BLK, 64), lambda i: (i, 0)),
        pl.BlockSpec((256, 128), lambda i: (0, 0)),
        pl.BlockSpec((128, 8), lambda i: (0, 0)),
    ],
    out_specs=[
        pl.BlockSpec((2, BLK, ROW), lambda i: (0, i, 0)),
        pl.BlockSpec((BLK, ADW), lambda i: (i, 0)),
    ],
    out_shape=[
        jax.ShapeDtypeStruct((2, N, ROW), jnp.float32),
        jax.ShapeDtypeStruct((N, ADW), jnp.float32),
    ],
)


def _sc_body(zs_hbm, ad_hbm, src_hbm, dst_hbm, zero_hbm, out_hbm,
             is0, is1, idxd_v, rows0, rows1, ad0, ad1,
             ex0_v, ex1_v, acc, si0, si1, ss0, ss1):
    c = lax.axis_index("c")
    s = lax.axis_index("s")
    idxs_b = (is0, is1)
    rows_b = (rows0, rows1)
    ad_b = (ad0, ad1)
    sem_i = (si0, si1)
    sem_s = (ss0, ss1)

    pltpu.sync_copy(zero_hbm, acc.at[pl.ds(s * RPS, RPS)])

    row0 = c * (E // CH) + s * NCHUNK
    pltpu.sync_copy(dst_hbm.at[pl.ds(row0, NCHUNK)], idxd_v)

    iota16 = lax.broadcasted_iota(jnp.int32, (16,), 0)
    col_s0 = jnp.broadcast_to(jnp.int32(2 * OUT), (16,))
    col_s1 = col_s0 + 1
    col_d0 = jnp.broadcast_to(2 * c, (16,))
    col_d1 = col_d0 + 1

    def copy_idxs(k, j):
        base = c * E + s * EPS + k * CH
        return pltpu.async_copy(src_hbm.at[pl.ds(base, CH)], idxs_b[j],
                                sem_i[j])

    def sync_gather(k, j):
        pltpu.sync_copy(zs_hbm.at[idxs_b[j]], rows_b[j])
        pltpu.sync_copy(ad_hbm.at[idxd_v.at[k]], ad_b[j])

    def issue_scatter(k, j):
        return pltpu.async_copy(rows_b[j], acc.at[idxd_v.at[k]], sem_s[j],
                                add=True)

    def compute(j):
        rows_v = rows_b[j]
        ad_v = ad_b[j]
        for g in range(CH // 16):
            ridx = iota16 + g * 16
            a_s0 = plsc.load_gather(rows_v, [ridx, col_s0])
            a_s1 = plsc.load_gather(rows_v, [ridx, col_s1])
            a_d0 = plsc.load_gather(ad_v, [ridx, col_d0])
            a_d1 = plsc.load_gather(ad_v, [ridx, col_d1])
            e0 = a_s0 + a_d0
            e1 = a_s1 + a_d1
            ex0 = jnp.exp(jnp.maximum(e0, 0.01 * e0))
            ex1 = jnp.exp(jnp.maximum(e1, 0.01 * e1))
            plsc.store_scatter(rows_v, [ridx, col_s0], ex0)
            plsc.store_scatter(rows_v, [ridx, col_s1], ex1)
            ex0_v[pl.ds(g * 16, 16)] = ex0
            ex1_v[pl.ds(g * 16, 16)] = ex1

        @pl.loop(0, CH)
        def _scale(i):
            bi = jnp.broadcast_to(i, (16,))
            w0 = plsc.load_gather(ex0_v, [bi])
            w1 = plsc.load_gather(ex1_v, [bi])
            for cb in range(4):
                rows_v[i, pl.ds(cb * 16, 16)] = rows_v[i, pl.ds(cb * 16, 16)] * w0
            for cb in range(4, 8):
                rows_v[i, pl.ds(cb * 16, 16)] = rows_v[i, pl.ds(cb * 16, 16)] * w1

    def body(k, j):
        q = 1 - j
        d_i = copy_idxs(k + 1, q)
        d_s = issue_scatter(k, j)
        d_i.wait()
        sync_gather(k + 1, q)
        compute(q)
        d_s.wait()

    copy_idxs(0, 0).wait()
    sync_gather(0, 0)
    plsc.subcore_barrier()
    compute(0)

    @pl.loop(0, NCHUNK - 1, step=2)
    def _chunks(k0):
        body(k0, 0)
        body(k0 + 1, 1)

    issue_scatter(NCHUNK - 1, 0).wait()

    plsc.subcore_barrier()
    pltpu.sync_copy(acc.at[pl.ds(s * RPS, RPS)],
                    out_hbm.at[c, pl.ds(s * RPS, RPS)])


_sc_params = pltpu.CompilerParams(use_tc_tiling_on_sc=False)
if "needs_layout_passes" in pltpu.CompilerParams.__dataclass_fields__:
    _sc_params = dataclasses.replace(_sc_params, needs_layout_passes=False)

_sc_edges = functools.partial(
    pl.kernel,
    out_type=jax.ShapeDtypeStruct((2, NPAD, ROW), jnp.float32),
    mesh=plsc.VectorSubcoreMesh(core_axis_name="c", subcore_axis_name="s"),
    compiler_params=_sc_params,
    scratch_types=[
        pltpu.VMEM((CH,), jnp.int32),
        pltpu.VMEM((CH,), jnp.int32),
        pltpu.VMEM((NCHUNK, CH), jnp.int32),
        pltpu.VMEM((CH, ROW), jnp.float32),
        pltpu.VMEM((CH, ROW), jnp.float32),
        pltpu.VMEM((CH, ADW), jnp.float32),
        pltpu.VMEM((CH, ADW), jnp.float32),
        pltpu.VMEM((CH,), jnp.float32),
        pltpu.VMEM((CH,), jnp.float32),
        pltpu.VMEM_SHARED((NPAD, ROW), jnp.float32),
        pltpu.SemaphoreType.DMA,
        pltpu.SemaphoreType.DMA,
        pltpu.SemaphoreType.DMA,
        pltpu.SemaphoreType.DMA,
    ],
)(_sc_body)


def _heads(acc, g):
    num = acc[g, :, 0:128]
    d0 = acc[g, :, 128]
    d1 = acc[g, :, 129]
    d0 = jnp.where(d0 > 0, d0, 1.0)
    d1 = jnp.where(d1 > 0, d1, 1.0)
    den = jnp.concatenate(
        [jnp.broadcast_to(d0[:, None], (BLK, OUT)),
         jnp.broadcast_to(d1[:, None], (BLK, OUT))], axis=1)
    return num / den


def _wsum_body(acc_ref, wp1_ref, bp1_ref, wp2_ref, wsum_ref):
    acc = acc_ref[...]
    h_f = _heads(acc, 0)
    h_l = _heads(acc, 1)

    def logit_sum(h):
        t = jnp.tanh(
            lax.dot_general(h, wp1_ref[...], (((1,), (1,)), ((), ())),
                            preferred_element_type=jnp.float32) + bp1_ref[...])
        w = lax.dot_general(t, wp2_ref[...], (((1,), (1,)), ((), ())),
                            preferred_element_type=jnp.float32)
        return jnp.sum(w)

    pf = logit_sum(h_f)
    plv = logit_sum(h_l)
    row = lax.broadcasted_iota(jnp.int32, (8, 128), 0)
    col = lax.broadcasted_iota(jnp.int32, (8, 128), 1)
    upd = (jnp.where((row == 0) & (col == 0), pf, 0.0)
           + jnp.where((row == 0) & (col == 1), plv, 0.0))

    @pl.when(pl.program_id(0) == 0)
    def _():
        wsum_ref[...] = jnp.zeros((8, 128), jnp.float32)

    wsum_ref[...] += upd


_wsum = pl.pallas_call(
    _wsum_body,
    grid=(N // BLK,),
    in_specs=[
        pl.BlockSpec((2, BLK, ROW), lambda i: (0, i, 0)),
        pl.BlockSpec((128, 128), lambda i: (0, 0)),
        pl.BlockSpec((1, 128), lambda i: (0, 0)),
        pl.BlockSpec((1, 128), lambda i: (0, 0)),
    ],
    out_specs=pl.BlockSpec((8, 128), lambda i: (0, 0)),
    out_shape=jax.ShapeDtypeStruct((8, 128), jnp.float32),
)


def _combine_body(acc_ref, wsum_ref, out_ref):
    s0 = wsum_ref[0, 0] / N
    s1 = wsum_ref[0, 1] / N
    m = jnp.maximum(s0, s1)
    b0 = jnp.exp(s0 - m)
    b1 = jnp.exp(s1 - m)
    beta0 = b0 / (b0 + b1)
    beta1 = b1 / (b0 + b1)
    acc = acc_ref[...]
    out_ref[...] = beta0 * _heads(acc, 0) + beta1 * _heads(acc, 1)


_combine = pl.pallas_call(
    _combine_body,
    grid=(N // BLK,),
    in_specs=[
        pl.BlockSpec((2, BLK, ROW), lambda i: (0, i, 0)),
        pl.BlockSpec((8, 128), lambda i: (0, 0)),
    ],
    out_specs=pl.BlockSpec((BLK, 128), lambda i: (i, 0)),
    out_shape=jax.ShapeDtypeStruct((N, 128), jnp.float32),
)


def kernel(embed_feat, h_gru, ei_follows, ei_likes, Win_f0, Win_f1, Wa_f0,
           Wa_f1, Win_l0, Win_l1, Wa_l0, Wa_l1, Wp1, bp1, Wp2):
    w_all = jnp.concatenate([Win_f0, Win_f1, Win_l0, Win_l1], axis=0)
    folds = []
    for win, wa in ((Win_f0, Wa_f0), (Win_f1, Wa_f1), (Win_l0, Wa_l0), (Win_l1, Wa_l1)):
        folds.append(win.T @ wa[0, :OUT])
    for win, wa in ((Win_f0, Wa_f0), (Win_f1, Wa_f1), (Win_l0, Wa_l0), (Win_l1, Wa_l1)):
        folds.append(win.T @ wa[0, OUT:])
    v_fold = jnp.stack(folds, axis=1)
    src_all = jnp.concatenate([ei_follows[0], ei_likes[0] + N])
    dst_all = jnp.concatenate([ei_follows[1], ei_likes[1]]).reshape(2 * E // CH, CH)
    zeros_h = jnp.zeros((RPS, ROW), jnp.float32)

    zs3, ad = _prep(embed_feat, h_gru, w_all, v_fold)
    zs_all = zs3.reshape(2 * N, ROW)
    acc3 = _sc_edges(zs_all, ad, src_all, dst_all, zeros_h)
    wsum = _wsum(acc3, Wp1, bp1.reshape(1, 128), Wp2)
    return _combine(acc3, wsum)
